# Initial kernel scaffold; baseline (speedup 1.0000x reference)
import jax
import jax.numpy as jnp
from jax.experimental import pallas as pl


def kernel(users, pos_items, neg_items, user_table, item_table, edge_index):
    raise NotImplementedError("write your pallas kernel here")


# speedup vs baseline: 6.0316x; 6.0316x over previous
import functools

import jax
import jax.numpy as jnp
from jax import lax
from jax.experimental import pallas as pl
from jax.experimental.pallas import tpu as pltpu
from jax.experimental.pallas import tpu_sc as plsc

NU = 25000
NI = 25000
NN = NU + NI
D = 64
NE = 800000
B = 4096

NC = 2
NS = 16
LN = 16

RPT = 1568
HALF = NS * RPT
NPAD = NC * HALF
DUMMY = HALF
ACC_ROWS = HALF + 8

CHUNK = 128
NCHUNK = 392
EPT = CHUNK * NCHUNK
EPAD = NS * EPT

_MESH = plsc.VectorSubcoreMesh(
    core_axis_name="c", subcore_axis_name="s", num_cores=NC, num_subcores=NS)


def _remap_dst(dst_v, idx_v, c):
    base = c * HALF
    for i in range(CHUNK // LN):
        d = dst_v[pl.ds(i * LN, LN)]
        loc = d - base
        ok = (loc >= 0) & (loc < HALF)
        idx_v[pl.ds(i * LN, LN)] = jnp.where(
            ok, loc, jnp.full((LN,), DUMMY, jnp.int32))


def _zero_rows(buf, nrows, ncol_groups):
    z = jnp.zeros((LN,), jnp.float32)

    def body(r, _):
        for g in range(ncol_groups):
            buf[r, pl.ds(g * LN, LN)] = z
        return 0

    lax.fori_loop(0, nrows, body, 0)


@functools.partial(
    pl.kernel,
    out_type=jax.ShapeDtypeStruct((NPAD, LN), jnp.float32),
    mesh=_MESH,
    compiler_params=pltpu.CompilerParams(use_tc_tiling_on_sc=False),
    scratch_types=[
        pltpu.VMEM((CHUNK,), jnp.int32),
        pltpu.VMEM((CHUNK,), jnp.int32),
        pltpu.VMEM((CHUNK, LN), jnp.float32),
        pltpu.VMEM_SHARED((ACC_ROWS, LN), jnp.float32),
    ],
)
def _prep(dst_hbm, deg16_hbm, dst_v, idx_v, ones_v, acc_sh):
    c = lax.axis_index("c")
    s = lax.axis_index("s")

    _zero_rows(ones_v, CHUNK, 1)
    for k in range(14):
        pltpu.sync_copy(ones_v.at[pl.ds(0, 112)],
                        acc_sh.at[pl.ds(s * RPT + k * 112, 112)])

    @pl.when(s == 0)
    def _():
        pltpu.sync_copy(ones_v.at[pl.ds(0, 8)], acc_sh.at[pl.ds(HALF, 8)])

    one = jnp.ones((LN,), jnp.float32)

    def fill(r, _):
        ones_v[r, pl.ds(0, LN)] = one
        return 0

    lax.fori_loop(0, CHUNK, fill, 0)
    plsc.subcore_barrier()

    def step(j, _):
        base = s * EPT + j * CHUNK
        pltpu.sync_copy(dst_hbm.at[pl.ds(base, CHUNK)], dst_v)
        _remap_dst(dst_v, idx_v, c)
        pltpu.sync_copy(ones_v, acc_sh.at[idx_v], add=True)
        return 0

    lax.fori_loop(0, NCHUNK, step, 0)
    plsc.subcore_barrier()

    pltpu.sync_copy(acc_sh.at[pl.ds(s * RPT, RPT)],
                    deg16_hbm.at[pl.ds(c * HALF + s * RPT, RPT)])


def _scale0_body(deg_ref, emb_ref, dinv_ref, s0_ref):
    dv = lax.rsqrt(jnp.maximum(deg_ref[...], 1.0))
    dinv_ref[...] = dv
    s0_ref[...] = emb_ref[...] * dv[:, :1]


_SC0_ROWS = 512


def _scale0(deg16, emb0):
    return pl.pallas_call(
        _scale0_body,
        grid=(NPAD // _SC0_ROWS,),
        in_specs=[
            pl.BlockSpec((_SC0_ROWS, LN), lambda i: (i, 0)),
            pl.BlockSpec((_SC0_ROWS, D), lambda i: (i, 0)),
        ],
        out_specs=[
            pl.BlockSpec((_SC0_ROWS, LN), lambda i: (i, 0)),
            pl.BlockSpec((_SC0_ROWS, D), lambda i: (i, 0)),
        ],
        out_shape=[
            jax.ShapeDtypeStruct((NPAD, LN), jnp.float32),
            jax.ShapeDtypeStruct((NPAD, D), jnp.float32),
        ],
    )(deg16, emb0)


@functools.partial(
    pl.kernel,
    out_type=jax.ShapeDtypeStruct((NPAD, D), jnp.float32),
    mesh=_MESH,
    compiler_params=pltpu.CompilerParams(use_tc_tiling_on_sc=False),
    scratch_types=[
        pltpu.VMEM((CHUNK,), jnp.int32),
        pltpu.VMEM((CHUNK,), jnp.int32),
        pltpu.VMEM((CHUNK,), jnp.int32),
        pltpu.VMEM((CHUNK, D), jnp.float32),
        pltpu.VMEM((LN, LN), jnp.float32),
        pltpu.VMEM((LN, D), jnp.float32),
        pltpu.VMEM_SHARED((ACC_ROWS, D), jnp.float32),
    ],
)
def _layer(scaled_hbm, src_hbm, dst_hbm, dinv16_hbm, out_hbm,
           src_v, dst_v, idx_v, rows_v, d16_v, out_v, acc_sh):
    c = lax.axis_index("c")
    s = lax.axis_index("s")

    _zero_rows(rows_v, CHUNK, D // LN)
    for k in range(14):
        pltpu.sync_copy(rows_v.at[pl.ds(0, 112)],
                        acc_sh.at[pl.ds(s * RPT + k * 112, 112)])

    @pl.when(s == 0)
    def _():
        pltpu.sync_copy(rows_v.at[pl.ds(0, 8)], acc_sh.at[pl.ds(HALF, 8)])

    plsc.subcore_barrier()

    def step(j, _):
        base = s * EPT + j * CHUNK
        pltpu.sync_copy(src_hbm.at[pl.ds(base, CHUNK)], src_v)
        pltpu.sync_copy(dst_hbm.at[pl.ds(base, CHUNK)], dst_v)
        pltpu.sync_copy(scaled_hbm.at[src_v], rows_v)
        _remap_dst(dst_v, idx_v, c)
        pltpu.sync_copy(rows_v, acc_sh.at[idx_v], add=True)
        return 0

    lax.fori_loop(0, NCHUNK, step, 0)
    plsc.subcore_barrier()

    def post(j, _):
        r0 = s * RPT + j * LN
        g0 = c * HALF + r0
        pltpu.sync_copy(acc_sh.at[pl.ds(r0, LN)], out_v)
        pltpu.sync_copy(dinv16_hbm.at[pl.ds(g0, LN)], d16_v)
        for i in range(LN):
            dv = d16_v[i, pl.ds(0, LN)]
            dv2 = dv * dv
            for g in range(D // LN):
                out_v[i, pl.ds(g * LN, LN)] = out_v[i, pl.ds(g * LN, LN)] * dv2
        pltpu.sync_copy(out_v, out_hbm.at[pl.ds(g0, LN)])
        return 0

    lax.fori_loop(0, RPT // LN, post, 0)


_BPT = B // (NC * NS)


@functools.partial(
    pl.kernel,
    out_type=(
        jax.ShapeDtypeStruct((B, D), jnp.float32),
        jax.ShapeDtypeStruct((B, D), jnp.float32),
        jax.ShapeDtypeStruct((B, D), jnp.float32),
    ),
    mesh=_MESH,
    compiler_params=pltpu.CompilerParams(use_tc_tiling_on_sc=False),
    scratch_types=[
        pltpu.VMEM((_BPT,), jnp.int32),
        pltpu.VMEM((_BPT,), jnp.int32),
        pltpu.VMEM((_BPT, D), jnp.float32),
        pltpu.VMEM((_BPT, D), jnp.float32),
        pltpu.VMEM((_BPT, D), jnp.float32),
        pltpu.VMEM((_BPT, D), jnp.float32),
        pltpu.VMEM((_BPT, LN), jnp.float32),
        pltpu.VMEM((_BPT, D), jnp.float32),
    ],
)
def _final(users_hbm, pos_hbm, neg_hbm, emb0_hbm, s1_hbm, s2_hbm, s3_hbm,
           dinv16_hbm, u_out, p_out, n_out,
           idx_v, gidx_v, base_v, r1_v, r2_v, r3_v, d16_v, out_v):
    c = lax.axis_index("c")
    s = lax.axis_index("s")
    w = s * NC + c
    quarter = jnp.full((LN,), 0.25, jnp.float32)

    for in_hbm, offset, out_hbm in (
            (users_hbm, 0, u_out), (pos_hbm, NU, p_out), (neg_hbm, NU, n_out)):
        base = w * _BPT
        pltpu.sync_copy(in_hbm.at[pl.ds(base, _BPT)], idx_v)
        if offset:
            off = jnp.full((LN,), offset, jnp.int32)
            for i in range(_BPT // LN):
                gidx_v[pl.ds(i * LN, LN)] = idx_v[pl.ds(i * LN, LN)] + off
            gi = gidx_v
        else:
            gi = idx_v
        pltpu.sync_copy(emb0_hbm.at[gi], base_v)
        pltpu.sync_copy(s1_hbm.at[gi], r1_v)
        pltpu.sync_copy(s2_hbm.at[gi], r2_v)
        pltpu.sync_copy(s3_hbm.at[gi], r3_v)
        pltpu.sync_copy(dinv16_hbm.at[gi], d16_v)

        def row(i, _):
            dv = d16_v[i, pl.ds(0, LN)]
            q = quarter / dv
            for g in range(D // LN):
                sl = pl.ds(g * LN, LN)
                sg = r1_v[i, sl] + r2_v[i, sl] + r3_v[i, sl]
                out_v[i, sl] = base_v[i, sl] * 0.25 + sg * q
            return 0

        lax.fori_loop(0, _BPT, row, 0)
        pltpu.sync_copy(out_v, out_hbm.at[pl.ds(base, _BPT)])


def kernel(users, pos_items, neg_items, user_table, item_table, edge_index):
    src = edge_index[0].astype(jnp.int32)
    dst = edge_index[1].astype(jnp.int32)
    pad_e = EPAD - NE
    srcp = jnp.concatenate([src, jnp.zeros((pad_e,), jnp.int32)])
    dstp = jnp.concatenate([dst, jnp.full((pad_e,), NPAD, jnp.int32)])
    emb0 = jnp.concatenate(
        [user_table, item_table, jnp.zeros((NPAD - NN, D), jnp.float32)], axis=0)

    deg16 = _prep(dstp)
    dinv16, scaled0 = _scale0(deg16, emb0)
    s1 = _layer(scaled0, srcp, dstp, dinv16)
    s2 = _layer(s1, srcp, dstp, dinv16)
    s3 = _layer(s2, srcp, dstp, dinv16)
    return _final(users, pos_items, neg_items, emb0, s1, s2, s3, dinv16)
